# baseline (speedup 1.0000x reference)
import functools

import jax
import jax.numpy as jnp
from jax import lax
from jax.experimental import pallas as pl
from jax.experimental.pallas import tpu as pltpu
from jax.experimental.pallas import tpu_sc as plsc


def _make_gather(B0, S, V, D, n_workers, nc):
    rows_per_w = B0 // n_workers
    CR = 16
    C = CR * S
    n_chunks = rows_per_w // CR
    mesh = plsc.VectorSubcoreMesh(core_axis_name="c", subcore_axis_name="s")

    @functools.partial(
        pl.kernel,
        mesh=mesh,
        compiler_params=pltpu.CompilerParams(use_tc_tiling_on_sc=False),
        out_type=jax.ShapeDtypeStruct((B0, S, D), jnp.float32),
    scratch_types=[
            pltpu.VMEM((rows_per_w, S), jnp.int32),
            pltpu.VMEM((C, D), jnp.float32),
            pltpu.VMEM((C, D), jnp.float32),
            pltpu.SemaphoreType.DMA,
            pltpu.SemaphoreType.DMA,
            pltpu.SemaphoreType.DMA,
            pltpu.SemaphoreType.DMA,
        ],
    )
    def k(ids_hbm, table_hbm, out_hbm, idx_all, rows0, rows1,
          semg0, semg1, sems0, sems1):
        wid = lax.axis_index("s") * nc + lax.axis_index("c")
        base = wid * rows_per_w
        pltpu.sync_copy(ids_hbm.at[pl.ds(base, rows_per_w)], idx_all)

        bufs = (rows0, rows1)
        gsems = (semg0, semg1)
        ssems = (sems0, sems1)

        def start_gathers(i):
            buf = bufs[i % 2]
            return [
                pltpu.async_copy(
                    table_hbm.at[idx_all.at[i * CR + j]],
                    buf.at[pl.ds(j * S, S)],
                    gsems[i % 2],
                )
                for j in range(CR)
            ]

        def start_stores(i):
            buf = bufs[i % 2]
            return [
                pltpu.async_copy(
                    buf.at[pl.ds(j * S, S)],
                    out_hbm.at[base + i * CR + j],
                    ssems[i % 2],
                )
                for j in range(CR)
            ]

        gcps = [None] * n_chunks
        scps = [None] * n_chunks
        gcps[0] = start_gathers(0)
        for i in range(n_chunks):
            for cp in gcps[i]:
                cp.wait()
            if i >= 1:
                for cp in scps[i - 1]:
                    cp.wait()
            if i + 1 < n_chunks:
                gcps[i + 1] = start_gathers(i + 1)
            scps[i] = start_stores(i)
        for cp in scps[n_chunks - 1]:
            cp.wait()

    return k


def kernel(entity_ids, table):
    B0, S = entity_ids.shape
    V, D = table.shape
    info = plsc.get_sparse_core_info()
    n_workers = info.num_cores * info.num_subcores
    ids = entity_ids.astype(jnp.int32)
    return _make_gather(B0, S, V, D, n_workers, info.num_cores)(ids, table)
